# baseline (speedup 1.0000x reference)
import functools

import jax
import jax.numpy as jnp
from jax import lax
from jax.experimental import pallas as pl
from jax.experimental.pallas import tpu as pltpu
from jax.experimental.pallas import tpu_sc as plsc

N = 10000
C = 128
K = 20
NEG_SLOPE = 0.2
EPS = 1e-5

NW = 32
NPAD = 10240
PW = NPAD // NW
CH = 2
NCH = PW // CH
CHROWS = CH * K

R_KNN = 400
R_MM = 1000


def _knn_body(posp_ref, post_ref, out_ref):
    i = pl.program_id(0)
    pb = posp_ref[...]
    pt = post_ref[...]
    sqb = jnp.sum(pb * pb, axis=1, keepdims=True)
    sqt = jnp.sum(pt * pt, axis=0, keepdims=True)
    d = sqb + sqt - 2.0 * jnp.dot(pb, pt, preferred_element_type=jnp.float32)
    colf = lax.broadcasted_iota(jnp.int32, (1, N), 1).astype(jnp.float32)
    rowf = lax.broadcasted_iota(jnp.int32, (R_KNN, 1), 0).astype(
        jnp.float32) + (i.astype(jnp.float32) * R_KNN)
    d = d + jnp.where(colf == rowf, 1e10, 0.0)

    m = jnp.min(d, axis=1, keepdims=True)
    colk = lax.broadcasted_iota(jnp.int32, (1, K), 1)
    picks0 = jnp.zeros((R_KNN, K), jnp.int32)

    def body(k, carry):
        dc, mc, picks = carry
        amf = jnp.min(jnp.where(dc == mc, colf, 1e9), axis=1, keepdims=True)
        picks = jnp.where(colk == k, amf.astype(jnp.int32), picks)
        dn = jnp.where(colf == amf, jnp.inf, dc)
        return dn, jnp.min(dn, axis=1, keepdims=True), picks

    _, _, picks = lax.fori_loop(0, K, body, (d, m, picks0))
    out_ref[...] = picks


def _knn(posp, post):
    return pl.pallas_call(
        _knn_body,
        grid=(N // R_KNN,),
        in_specs=[
            pl.BlockSpec((R_KNN, 8), lambda i: (i, 0)),
            pl.BlockSpec((8, N), lambda i: (0, 0)),
        ],
        out_specs=pl.BlockSpec((R_KNN, K), lambda i: (i, 0)),
        out_shape=jax.ShapeDtypeStruct((N, K), jnp.int32),
    )(posp, post)


def _ab_body(x_ref, w_ref, b_ref, a_ref, bm_ref):
    xb = x_ref[...]
    w1 = w_ref[0:C, :]
    w2 = w_ref[C:2 * C, :]
    a_ref[...] = jnp.dot(xb, w1 - w2,
                         preferred_element_type=jnp.float32) + b_ref[...]
    bm_ref[...] = jnp.dot(xb, w2, preferred_element_type=jnp.float32)


def _ab(x, W, b2d):
    return pl.pallas_call(
        _ab_body,
        grid=(N // R_MM,),
        in_specs=[
            pl.BlockSpec((R_MM, C), lambda i: (i, 0)),
            pl.BlockSpec((2 * C, C), lambda i: (0, 0)),
            pl.BlockSpec((1, C), lambda i: (0, 0)),
        ],
        out_specs=[
            pl.BlockSpec((R_MM, C), lambda i: (i, 0)),
            pl.BlockSpec((R_MM, C), lambda i: (i, 0)),
        ],
        out_shape=[
            jax.ShapeDtypeStruct((N, C), jnp.float32),
            jax.ShapeDtypeStruct((N, C), jnp.float32),
        ],
    )(x, W, b2d)


def _sc_gather_stats(bmat, idx_flat):
    mesh = plsc.VectorSubcoreMesh(core_axis_name="c", subcore_axis_name="s")

    @functools.partial(
        pl.kernel,
        mesh=mesh,
        out_type=jax.ShapeDtypeStruct((4, NPAD, C), jnp.float32),
        scratch_types=[
            pltpu.VMEM((PW * K,), jnp.int32),
            pltpu.VMEM((2, CHROWS, C), jnp.float32),
            pltpu.VMEM((4, CH, C), jnp.float32),
            pltpu.SemaphoreType.DMA,
        ],
    )
    def sck(b_hbm, idx_hbm, out_hbm, idx_v, rows_v, res_v, sem):
        wid = lax.axis_index("s") * 2 + lax.axis_index("c")
        ibase = wid * (PW * K)
        rbase = wid * PW
        pltpu.sync_copy(idx_hbm.at[pl.ds(ibase, PW * K)], idx_v)
        pltpu.async_copy(
            b_hbm.at[idx_v.at[pl.ds(0, CHROWS)]], rows_v.at[0], sem).wait()

        def half(bank, ci):
            nci = jnp.minimum(ci + 1, NCH - 1)
            h = pltpu.async_copy(
                b_hbm.at[idx_v.at[pl.ds(nci * CHROWS, CHROWS)]],
                rows_v.at[1 - bank], sem)
            for p in range(CH):
                for c in range(C // 16):
                    sl = pl.ds(c * 16, 16)
                    v = rows_v[bank, p * K, sl]
                    mx = v
                    mn = v
                    sm = v
                    sq = v * v
                    for kk in range(1, K):
                        v = rows_v[bank, p * K + kk, sl]
                        mx = jnp.maximum(mx, v)
                        mn = jnp.minimum(mn, v)
                        sm = sm + v
                        sq = sq + v * v
                    res_v[0, p, sl] = mx
                    res_v[1, p, sl] = mn
                    res_v[2, p, sl] = sm
                    res_v[3, p, sl] = sq
            pltpu.sync_copy(res_v,
                            out_hbm.at[:, pl.ds(rbase + ci * CH, CH)])
            h.wait()

        def body(t, carry):
            half(0, 2 * t)
            half(1, 2 * t + 1)
            return carry

        lax.fori_loop(0, NCH // 2, body, jnp.int32(0))

    return sck(bmat, idx_flat)


def _stats_body(a_ref, gsum_ref, gsq_ref, out_ref):
    i = pl.program_id(0)
    a = a_ref[...]
    s = gsum_ref[...]
    q = gsq_ref[...]
    z = jnp.zeros((3, C), jnp.float32)
    vals = jnp.concatenate([
        jnp.sum(a, axis=0, keepdims=True),
        jnp.sum(a * a, axis=0, keepdims=True),
        jnp.sum(s, axis=0, keepdims=True),
        jnp.sum(q, axis=0, keepdims=True),
        jnp.sum(a * s, axis=0, keepdims=True),
        z,
    ], axis=0)

    @pl.when(i == 0)
    def _():
        out_ref[...] = vals

    @pl.when(i != 0)
    def _():
        out_ref[...] = out_ref[...] + vals


def _stats(a, gsum, gsq):
    return pl.pallas_call(
        _stats_body,
        grid=(N // R_MM,),
        in_specs=[
            pl.BlockSpec((R_MM, C), lambda i: (i, 0)),
            pl.BlockSpec((R_MM, C), lambda i: (i, 0)),
            pl.BlockSpec((R_MM, C), lambda i: (i, 0)),
        ],
        out_specs=pl.BlockSpec((8, C), lambda i: (0, 0)),
        out_shape=jax.ShapeDtypeStruct((8, C), jnp.float32),
    )(a, gsum, gsq)


def _final_body(a_ref, gmax_ref, gmin_ref, st_ref, gam_ref, bet_ref, o_ref):
    nk = jnp.float32(N * K)
    sum_a = st_ref[0:1, :]
    sum_a2 = st_ref[1:2, :]
    tot_b = st_ref[2:3, :]
    tot_b2 = st_ref[3:4, :]
    cross = st_ref[4:5, :]
    mean = (K * sum_a + tot_b) / nk
    e2 = (K * sum_a2 + 2.0 * cross + tot_b2) / nk
    var = e2 - mean * mean
    s = gam_ref[...] / jnp.sqrt(var + EPS)
    t0 = bet_ref[...] - mean * s
    a = a_ref[...]
    hmax = a + gmax_ref[...]
    hmin = a + gmin_ref[...]
    h = jnp.where(s >= 0.0, hmax, hmin) * s + t0
    o_ref[...] = jnp.where(h >= 0.0, h, NEG_SLOPE * h)


def _final(a, gmax, gmin, st, gam, bet):
    return pl.pallas_call(
        _final_body,
        grid=(N // R_MM,),
        in_specs=[
            pl.BlockSpec((R_MM, C), lambda i: (i, 0)),
            pl.BlockSpec((R_MM, C), lambda i: (i, 0)),
            pl.BlockSpec((R_MM, C), lambda i: (i, 0)),
            pl.BlockSpec((8, C), lambda i: (0, 0)),
            pl.BlockSpec((1, C), lambda i: (0, 0)),
            pl.BlockSpec((1, C), lambda i: (0, 0)),
        ],
        out_specs=pl.BlockSpec((R_MM, C), lambda i: (i, 0)),
        out_shape=jax.ShapeDtypeStruct((N, C), jnp.float32),
    )(a, gmax, gmin, st, gam, bet)


def kernel(x, pos, W, b, gamma, beta):
    posp = jnp.concatenate(
        [pos, jnp.zeros((N, 5), jnp.float32)], axis=1)
    post = posp.T
    idx = _knn(posp, post)

    a, bmat = _ab(x, W, b.reshape(1, C))

    pad = (jnp.arange((NPAD - N) * K, dtype=jnp.int32) % N).reshape(
        NPAD - N, K)
    idx_flat = jnp.concatenate([idx, pad], axis=0).reshape(NPAD * K)

    red = _sc_gather_stats(bmat, idx_flat)
    gmax = red[0, :N]
    gmin = red[1, :N]
    gsum = red[2, :N]
    gsq = red[3, :N]

    st = _stats(a, gsum, gsq)
    return _final(a, gmax, gmin, st,
                  gamma.reshape(1, C), beta.reshape(1, C))


# speedup vs baseline: 1.0028x; 1.0028x over previous
import functools

import jax
import jax.numpy as jnp
from jax import lax
from jax.experimental import pallas as pl
from jax.experimental.pallas import tpu as pltpu
from jax.experimental.pallas import tpu_sc as plsc

N = 10000
C = 128
K = 20
NEG_SLOPE = 0.2
EPS = 1e-5

NW = 32
NPAD = 10240
PW = NPAD // NW
CH = 2
NCH = PW // CH
CHROWS = CH * K

R_KNN = 200
R_MM = 1000


def _knn_body(posp_ref, post_ref, out_ref):
    i = pl.program_id(0)
    pb = posp_ref[...]
    pt = post_ref[...]
    sqb = jnp.sum(pb * pb, axis=1, keepdims=True)
    sqt = jnp.sum(pt * pt, axis=0, keepdims=True)
    d = sqb + sqt - 2.0 * jnp.dot(pb, pt, preferred_element_type=jnp.float32)
    colf = lax.broadcasted_iota(jnp.int32, (1, N), 1).astype(jnp.float32)
    rowf = lax.broadcasted_iota(jnp.int32, (R_KNN, 1), 0).astype(
        jnp.float32) + (i.astype(jnp.float32) * R_KNN)
    d = d + jnp.where(colf == rowf, 1e10, 0.0)

    m = jnp.min(d, axis=1, keepdims=True)
    colk = lax.broadcasted_iota(jnp.int32, (1, K), 1)
    picks0 = jnp.zeros((R_KNN, K), jnp.int32)

    def body(k, carry):
        dc, mc, picks = carry
        amf = jnp.min(jnp.where(dc == mc, colf, 1e9), axis=1, keepdims=True)
        picks = jnp.where(colk == k, amf.astype(jnp.int32), picks)
        dn = jnp.where(colf == amf, jnp.inf, dc)
        return dn, jnp.min(dn, axis=1, keepdims=True), picks

    _, _, picks = lax.fori_loop(0, K, body, (d, m, picks0))
    out_ref[...] = picks


def _knn(posp, post):
    return pl.pallas_call(
        _knn_body,
        grid=(N // R_KNN,),
        in_specs=[
            pl.BlockSpec((R_KNN, 8), lambda i: (i, 0)),
            pl.BlockSpec((8, N), lambda i: (0, 0)),
        ],
        out_specs=pl.BlockSpec((R_KNN, K), lambda i: (i, 0)),
        out_shape=jax.ShapeDtypeStruct((N, K), jnp.int32),
    )(posp, post)


def _ab_body(x_ref, w_ref, b_ref, a_ref, bm_ref):
    xb = x_ref[...]
    w1 = w_ref[0:C, :]
    w2 = w_ref[C:2 * C, :]
    a_ref[...] = jnp.dot(xb, w1 - w2,
                         preferred_element_type=jnp.float32) + b_ref[...]
    bm_ref[...] = jnp.dot(xb, w2, preferred_element_type=jnp.float32)


def _ab(x, W, b2d):
    return pl.pallas_call(
        _ab_body,
        grid=(N // R_MM,),
        in_specs=[
            pl.BlockSpec((R_MM, C), lambda i: (i, 0)),
            pl.BlockSpec((2 * C, C), lambda i: (0, 0)),
            pl.BlockSpec((1, C), lambda i: (0, 0)),
        ],
        out_specs=[
            pl.BlockSpec((R_MM, C), lambda i: (i, 0)),
            pl.BlockSpec((R_MM, C), lambda i: (i, 0)),
        ],
        out_shape=[
            jax.ShapeDtypeStruct((N, C), jnp.float32),
            jax.ShapeDtypeStruct((N, C), jnp.float32),
        ],
    )(x, W, b2d)


def _sc_gather_stats(bmat, idx_flat):
    mesh = plsc.VectorSubcoreMesh(core_axis_name="c", subcore_axis_name="s")

    @functools.partial(
        pl.kernel,
        mesh=mesh,
        out_type=jax.ShapeDtypeStruct((4, NPAD, C), jnp.float32),
        scratch_types=[
            pltpu.VMEM((PW * K,), jnp.int32),
            pltpu.VMEM((2, CHROWS, C), jnp.float32),
            pltpu.VMEM((4, CH, C), jnp.float32),
            pltpu.SemaphoreType.DMA,
        ],
    )
    def sck(b_hbm, idx_hbm, out_hbm, idx_v, rows_v, res_v, sem):
        wid = lax.axis_index("s") * 2 + lax.axis_index("c")
        ibase = wid * (PW * K)
        rbase = wid * PW
        pltpu.sync_copy(idx_hbm.at[pl.ds(ibase, PW * K)], idx_v)
        pltpu.async_copy(
            b_hbm.at[idx_v.at[pl.ds(0, CHROWS)]], rows_v.at[0], sem).wait()

        def half(bank, ci):
            nci = jnp.minimum(ci + 1, NCH - 1)
            h = pltpu.async_copy(
                b_hbm.at[idx_v.at[pl.ds(nci * CHROWS, CHROWS)]],
                rows_v.at[1 - bank], sem)
            for p in range(CH):
                for c in range(C // 16):
                    sl = pl.ds(c * 16, 16)
                    v = rows_v[bank, p * K, sl]
                    mx = v
                    mn = v
                    sm = v
                    sq = v * v
                    for kk in range(1, K):
                        v = rows_v[bank, p * K + kk, sl]
                        mx = jnp.maximum(mx, v)
                        mn = jnp.minimum(mn, v)
                        sm = sm + v
                        sq = sq + v * v
                    res_v[0, p, sl] = mx
                    res_v[1, p, sl] = mn
                    res_v[2, p, sl] = sm
                    res_v[3, p, sl] = sq
            pltpu.sync_copy(res_v,
                            out_hbm.at[:, pl.ds(rbase + ci * CH, CH)])
            h.wait()

        def body(t, carry):
            half(0, 2 * t)
            half(1, 2 * t + 1)
            return carry

        lax.fori_loop(0, NCH // 2, body, jnp.int32(0))

    return sck(bmat, idx_flat)


def _stats_body(a_ref, gsum_ref, gsq_ref, out_ref):
    i = pl.program_id(0)
    a = a_ref[...]
    s = gsum_ref[...]
    q = gsq_ref[...]
    z = jnp.zeros((3, C), jnp.float32)
    vals = jnp.concatenate([
        jnp.sum(a, axis=0, keepdims=True),
        jnp.sum(a * a, axis=0, keepdims=True),
        jnp.sum(s, axis=0, keepdims=True),
        jnp.sum(q, axis=0, keepdims=True),
        jnp.sum(a * s, axis=0, keepdims=True),
        z,
    ], axis=0)

    @pl.when(i == 0)
    def _():
        out_ref[...] = vals

    @pl.when(i != 0)
    def _():
        out_ref[...] = out_ref[...] + vals


def _stats(a, gsum, gsq):
    return pl.pallas_call(
        _stats_body,
        grid=(N // R_MM,),
        in_specs=[
            pl.BlockSpec((R_MM, C), lambda i: (i, 0)),
            pl.BlockSpec((R_MM, C), lambda i: (i, 0)),
            pl.BlockSpec((R_MM, C), lambda i: (i, 0)),
        ],
        out_specs=pl.BlockSpec((8, C), lambda i: (0, 0)),
        out_shape=jax.ShapeDtypeStruct((8, C), jnp.float32),
    )(a, gsum, gsq)


def _final_body(a_ref, gmax_ref, gmin_ref, st_ref, gam_ref, bet_ref, o_ref):
    nk = jnp.float32(N * K)
    sum_a = st_ref[0:1, :]
    sum_a2 = st_ref[1:2, :]
    tot_b = st_ref[2:3, :]
    tot_b2 = st_ref[3:4, :]
    cross = st_ref[4:5, :]
    mean = (K * sum_a + tot_b) / nk
    e2 = (K * sum_a2 + 2.0 * cross + tot_b2) / nk
    var = e2 - mean * mean
    s = gam_ref[...] / jnp.sqrt(var + EPS)
    t0 = bet_ref[...] - mean * s
    a = a_ref[...]
    hmax = a + gmax_ref[...]
    hmin = a + gmin_ref[...]
    h = jnp.where(s >= 0.0, hmax, hmin) * s + t0
    o_ref[...] = jnp.where(h >= 0.0, h, NEG_SLOPE * h)


def _final(a, gmax, gmin, st, gam, bet):
    return pl.pallas_call(
        _final_body,
        grid=(N // R_MM,),
        in_specs=[
            pl.BlockSpec((R_MM, C), lambda i: (i, 0)),
            pl.BlockSpec((R_MM, C), lambda i: (i, 0)),
            pl.BlockSpec((R_MM, C), lambda i: (i, 0)),
            pl.BlockSpec((8, C), lambda i: (0, 0)),
            pl.BlockSpec((1, C), lambda i: (0, 0)),
            pl.BlockSpec((1, C), lambda i: (0, 0)),
        ],
        out_specs=pl.BlockSpec((R_MM, C), lambda i: (i, 0)),
        out_shape=jax.ShapeDtypeStruct((N, C), jnp.float32),
    )(a, gmax, gmin, st, gam, bet)


def kernel(x, pos, W, b, gamma, beta):
    posp = jnp.concatenate(
        [pos, jnp.zeros((N, 5), jnp.float32)], axis=1)
    post = posp.T
    idx = _knn(posp, post)

    a, bmat = _ab(x, W, b.reshape(1, C))

    pad = (jnp.arange((NPAD - N) * K, dtype=jnp.int32) % N).reshape(
        NPAD - N, K)
    idx_flat = jnp.concatenate([idx, pad], axis=0).reshape(NPAD * K)

    red = _sc_gather_stats(bmat, idx_flat)
    gmax = red[0, :N]
    gmin = red[1, :N]
    gsum = red[2, :N]
    gsq = red[3, :N]

    st = _stats(a, gsum, gsq)
    return _final(a, gmax, gmin, st,
                  gamma.reshape(1, C), beta.reshape(1, C))
